# baseline (speedup 1.0000x reference)
import jax
import jax.numpy as jnp
from jax import lax
from jax.experimental import pallas as pl
from jax.experimental.pallas import tpu as pltpu
from jax.experimental.pallas import tpu_sc as plsc

VOCAB = 1_000_000
D = 64
BATCH = 4096
SEQ = 50

_info = plsc.get_sparse_core_info()
NC = _info.num_cores
NS = _info.num_subcores
NW = NC * NS
CH = BATCH // NW
NB = 2

NFULL = VOCAB // 128
NTAIL = VOCAB - NFULL * 128
KMAX = (NFULL + NW - 1) // NW


def _make_convert():
  mesh = plsc.VectorSubcoreMesh(core_axis_name="c", subcore_axis_name="s")

  @pl.kernel(
      out_type=jax.ShapeDtypeStruct((VOCAB, D), jnp.float32),
      mesh=mesh,
      compiler_params=pltpu.CompilerParams(needs_layout_passes=False),
      scratch_types=(
          [pltpu.VMEM((D, 136), jnp.float32) for _ in range(NB)]
          + [pltpu.VMEM((128, D), jnp.float32) for _ in range(NB)]
          + [pltpu.SemaphoreType.DMA for _ in range(2 * NB)]
      ),
  )
  def convert(tt_hbm, tail_hbm, tp_hbm, *bufs_sems):
    ibufs = bufs_sems[:NB]
    obufs = bufs_sems[NB:2 * NB]
    si = bufs_sems[2 * NB:3 * NB]
    so = bufs_sems[3 * NB:4 * NB]
    wid = lax.axis_index("s") * NC + lax.axis_index("c")
    iota = lax.iota(jnp.int32, 16)

    def blk(k):
      return wid + NW * k

    def start_in(k, b):
      pltpu.async_copy(
          tt_hbm.at[:, pl.ds(blk(k) * 128, 128)],
          ibufs[b].at[:, pl.ds(0, 128)], si[b])

    def transpose_block(b):
      @pl.loop(0, 128, unroll=4)
      def _r(r):
        col = jnp.broadcast_to(r, (16,))
        for kk in range(D // 16):
          obufs[b][r, pl.ds(16 * kk, 16)] = plsc.load_gather(
              ibufs[b], [16 * kk + iota, col])

    for b in range(NB):
      @pl.when(blk(b) < NFULL)
      def _():
        start_in(b, b)

    @pl.loop(0, KMAX, step=NB)
    def _blocks(k0):
      for b in range(NB):
        k = k0 + b

        @pl.when(blk(k) < NFULL)
        def _():
          pltpu.make_async_copy(
              tt_hbm.at[:, pl.ds(0, 128)], ibufs[b].at[:, pl.ds(0, 128)],
              si[b]).wait()

          @pl.when(k >= NB)
          def _():
            pltpu.make_async_copy(
                obufs[b], tp_hbm.at[pl.ds(0, 128), :], so[b]).wait()

          transpose_block(b)
          pltpu.async_copy(
              obufs[b], tp_hbm.at[pl.ds(blk(k) * 128, 128), :], so[b])

          @pl.when(blk(k + NB) < NFULL)
          def _():
            start_in(k + NB, b)

    for b in range(NB):
      pltpu.make_async_copy(
          obufs[b], tp_hbm.at[pl.ds(0, 128), :], so[b]).wait()

    @pl.when(wid == 0)
    def _():
      pltpu.sync_copy(tail_hbm, tp_hbm.at[pl.ds(NFULL * 128, NTAIL), :])

  return convert


def _make_lookup():
  mesh = plsc.VectorSubcoreMesh(core_axis_name="c", subcore_axis_name="s")

  @pl.kernel(
      out_type=jax.ShapeDtypeStruct((NW, CH, SEQ, D), jnp.float32),
      mesh=mesh,
      scratch_types=(
          [pltpu.VMEM((SEQ, CH), jnp.int32)]
          + [pltpu.VMEM((CH, D), jnp.float32) for _ in range(NB)]
          + [pltpu.SemaphoreType.DMA for _ in range(2 * NB)]
      ),
  )
  def lookup(tp_hbm, xt_hbm, out_hbm, idx_v, *bufs_sems):
    gbufs = bufs_sems[:NB]
    sg = bufs_sems[NB:2 * NB]
    sw = bufs_sems[2 * NB:3 * NB]
    wid = lax.axis_index("s") * NC + lax.axis_index("c")
    b0 = wid * CH
    pltpu.sync_copy(xt_hbm.at[:, pl.ds(b0, CH)], idx_v)

    def rowdma(s, b):
      @pl.loop(0, CH // 16)
      def _rows(g):
        vv = idx_v[s, pl.ds(g * 16, 16)]
        for l in range(16):
          pltpu.async_copy(
              tp_hbm.at[vv[l]], gbufs[b].at[g * 16 + l], sg[b])

    def out_slice(s):
      return out_hbm.at[wid, :, s, :]

    rowdma(0, 0)

    @pl.loop(0, SEQ, step=NB)
    def _chunks(s0):
      for b in range(NB):
        s = s0 + b
        b2 = (b + 1) % NB

        @pl.when(s + 1 < SEQ)
        def _():
          @pl.when(s >= 1)
          def _():
            pltpu.make_async_copy(gbufs[b2], out_slice(s - 1), sw[b2]).wait()
          rowdma(s + 1, b2)

        pltpu.make_async_copy(
            tp_hbm.at[pl.ds(0, CH), :], gbufs[b], sg[b]).wait()
        pltpu.async_copy(gbufs[b], out_slice(s), sw[b])

    for b in range(NB):
      s = SEQ - NB + b
      pltpu.make_async_copy(gbufs[b], out_slice(s), sw[b]).wait()

  return lookup


_convert = _make_convert()
_lookup = _make_lookup()


@jax.jit
def kernel(x, table):
  tailp = table[NFULL * 128:]
  tp = _convert(table.T, tailp)
  o4 = _lookup(tp, x.T.astype(jnp.int32))
  return o4.reshape(BATCH, SEQ, D)


# speedup vs baseline: 4.5978x; 4.5978x over previous
import jax
import jax.numpy as jnp
from jax import lax
from jax.experimental import pallas as pl
from jax.experimental.pallas import tpu as pltpu
from jax.experimental.pallas import tpu_sc as plsc

VOCAB = 1_000_000
D = 64
BATCH = 4096
SEQ = 50

_info = plsc.get_sparse_core_info()
NC = _info.num_cores
NS = _info.num_subcores
NW = NC * NS
CH = BATCH // NW
NB = 2


def _make_lookup():
  mesh = plsc.VectorSubcoreMesh(core_axis_name="c", subcore_axis_name="s")

  @pl.kernel(
      out_type=jax.ShapeDtypeStruct((NW, CH, SEQ, D), jnp.float32),
      mesh=mesh,
      scratch_types=(
          [pltpu.VMEM((SEQ, CH), jnp.int32)]
          + [pltpu.VMEM((CH, D), jnp.float32) for _ in range(NB)]
          + [pltpu.SemaphoreType.DMA for _ in range(2 * NB)]
      ),
  )
  def lookup(t_hbm, xt_hbm, out_hbm, idx_v, *bufs_sems):
    gbufs = bufs_sems[:NB]
    sg = bufs_sems[NB:2 * NB]
    sw = bufs_sems[2 * NB:3 * NB]
    wid = lax.axis_index("s") * NC + lax.axis_index("c")
    b0 = wid * CH
    pltpu.sync_copy(xt_hbm.at[:, pl.ds(b0, CH)], idx_v)

    def rowdma(s, b):
      @pl.loop(0, CH // 16)
      def _rows(g):
        vv = idx_v[s, pl.ds(g * 16, 16)]
        hh = lax.div(vv, VOCAB // 4)
        ll = lax.rem(vv, VOCAB // 4)
        for l in range(16):
          pltpu.async_copy(
              t_hbm.at[hh[l], ll[l]], gbufs[b].at[g * 16 + l], sg[b])

    def out_slice(s):
      return out_hbm.at[wid, :, s, :]

    rowdma(0, 0)

    @pl.loop(0, SEQ, step=NB)
    def _chunks(s0):
      for b in range(NB):
        s = s0 + b
        b2 = (b + 1) % NB

        @pl.when(s + 1 < SEQ)
        def _():
          @pl.when(s >= 1)
          def _():
            pltpu.make_async_copy(gbufs[b2], out_slice(s - 1), sw[b2]).wait()
          rowdma(s + 1, b2)

        pltpu.make_async_copy(
            t_hbm.at[0, pl.ds(0, CH), :], gbufs[b], sg[b]).wait()
        pltpu.async_copy(gbufs[b], out_slice(s), sw[b])

    for b in range(NB):
      s = SEQ - NB + b
      pltpu.make_async_copy(gbufs[b], out_slice(s), sw[b]).wait()

  return lookup


_lookup = _make_lookup()


@jax.jit
def kernel(x, table):
  t4 = table.reshape(4, VOCAB // 4, D)
  o4 = _lookup(t4, x.T.astype(jnp.int32))
  return o4.reshape(BATCH, SEQ, D)
